# baseline (speedup 1.0000x reference)
import functools

import jax
import jax.numpy as jnp
from jax import lax
from jax.experimental import pallas as pl
from jax.experimental.pallas import tpu as pltpu
from jax.experimental.pallas import tpu_sc as plsc

_C = 1000
_N = 16384
_NC = 2
_NS = 16
_NW = _NC * _NS
_ROWS_PER_W = _N // _NW
_R = 64
_CHUNKS = _ROWS_PER_W // _R
_L = 16


def _onehot_body(x_hbm, out_hbm, idx_v, buf_v):
    cid = lax.axis_index("c")
    sid = lax.axis_index("s")
    wid = sid * _NC + cid
    base_row = wid * _ROWS_PER_W

    zeros16 = jnp.zeros((_L,), jnp.int32)
    ones16 = jnp.ones((_L,), jnp.int32)
    lane = lax.iota(jnp.int32, _L)

    def _zf(i, carry):
        base = pl.multiple_of(i * (16 * _L), 16 * _L)
        for u in range(16):
            buf_v[pl.ds(base + u * _L, _L)] = zeros16
        return carry

    lax.fori_loop(0, _R * _C // (16 * _L), _zf, 0)

    for g in range(_CHUNKS):
        row0 = base_row + g * _R
        pltpu.sync_copy(x_hbm.at[pl.ds(row0, _R)], idx_v)
        for j in range(_R // _L):
            xv = idx_v[pl.ds(j * _L, _L)]
            off = (lane + (j * _L)) * _C + xv
            plsc.store_scatter(buf_v, [off], ones16)
        pltpu.sync_copy(buf_v, out_hbm.at[pl.ds(row0 * _C, _R * _C)])
        for j in range(_R // _L):
            xv = idx_v[pl.ds(j * _L, _L)]
            off = (lane + (j * _L)) * _C + xv
            plsc.store_scatter(buf_v, [off], zeros16)


_onehot_sc = functools.partial(
    pl.kernel,
    out_type=jax.ShapeDtypeStruct((_N * _C,), jnp.int32),
    mesh=plsc.VectorSubcoreMesh(
        core_axis_name="c", subcore_axis_name="s",
        num_cores=_NC, num_subcores=_NS,
    ),
    scratch_types=[
        pltpu.VMEM((_R,), jnp.int32),
        pltpu.VMEM((_R * _C,), jnp.int32),
    ],
    compiler_params=pltpu.CompilerParams(needs_layout_passes=False),
)(_onehot_body)


def kernel(x):
    return _onehot_sc(x).reshape(_N, _C)


# speedup vs baseline: 1.5846x; 1.5846x over previous
import functools

import jax
import jax.numpy as jnp
from jax import lax
from jax.experimental import pallas as pl
from jax.experimental.pallas import tpu as pltpu
from jax.experimental.pallas import tpu_sc as plsc

_C = 1000
_N = 16384
_NC = 2
_NS = 16
_NW = _NC * _NS
_ROWS_PER_W = _N // _NW
_R = 64
_CHUNKS = _ROWS_PER_W // _R
_L = 16


def _onehot_body(x_hbm, out_hbm, idx_v, buf_v):
    cid = lax.axis_index("c")
    sid = lax.axis_index("s")
    wid = sid * _NC + cid
    base_row = wid * _ROWS_PER_W

    zeros16 = jnp.zeros((_L,), jnp.int32)
    ones16 = jnp.ones((_L,), jnp.int32)
    lane = lax.iota(jnp.int32, _L)

    def _zrow(r, carry):
        for u in range(_C // _L):
            buf_v[r, pl.ds(u * _L, _L)] = zeros16
        buf_v[r, pl.ds(_C - _L, _L)] = zeros16
        return carry

    lax.fori_loop(0, _R, _zrow, 0)

    for g in range(_CHUNKS):
        row0 = base_row + g * _R
        pltpu.sync_copy(x_hbm.at[pl.ds(row0, _R)], idx_v)
        for j in range(_R // _L):
            xv = idx_v[pl.ds(j * _L, _L)]
            rows = lane + (j * _L)
            plsc.store_scatter(buf_v, [rows, xv], ones16)
        pltpu.sync_copy(buf_v, out_hbm.at[pl.ds(row0, _R), :])
        for j in range(_R // _L):
            xv = idx_v[pl.ds(j * _L, _L)]
            rows = lane + (j * _L)
            plsc.store_scatter(buf_v, [rows, xv], zeros16)


_onehot_sc = functools.partial(
    pl.kernel,
    out_type=jax.ShapeDtypeStruct((_N, _C), jnp.int32),
    mesh=plsc.VectorSubcoreMesh(
        core_axis_name="c", subcore_axis_name="s",
        num_cores=_NC, num_subcores=_NS,
    ),
    scratch_types=[
        pltpu.VMEM((_R,), jnp.int32),
        pltpu.VMEM((_R, _C), jnp.int32),
    ],
    compiler_params=pltpu.CompilerParams(needs_layout_passes=False),
)(_onehot_body)


def kernel(x):
    return _onehot_sc(x)


# speedup vs baseline: 2.7672x; 1.7464x over previous
import functools

import jax
import jax.numpy as jnp
from jax import lax
from jax.experimental import pallas as pl
from jax.experimental.pallas import tpu as pltpu
from jax.experimental.pallas import tpu_sc as plsc

_C = 1000
_N = 16384
_NC = 2
_NS = 16
_NW = _NC * _NS
_CPW = 32
_CHUNK = 1024
_NCHUNKS = _N // _CHUNK
_L = 16
_NBUF = 2
_NIBUF = 4


def _onehot_body(x_hbm, out_hbm, buf0, buf1, ib0, ib1, ib2, ib3,
                 so0, so1, si0, si1, si2, si3):
    bufs = [buf0, buf1]
    ibufs = [ib0, ib1, ib2, ib3]
    sem_o = [so0, so1]
    sem_i = [si0, si1, si2, si3]

    cid = lax.axis_index("c")
    sid = lax.axis_index("s")
    wid = sid * _NC + cid
    lo = wid * _CPW

    zeros16 = jnp.zeros((_L,), jnp.int32)
    ones16 = jnp.ones((_L,), jnp.int32)
    lane = lax.iota(jnp.int32, _L)

    def idx_dma(g, ib):
        return pltpu.async_copy(
            x_hbm.at[pl.ds(g * _CHUNK, _CHUNK)], ibufs[ib], sem_i[ib])

    idescs = {}
    for g in range(min(_NIBUF, _NCHUNKS)):
        idescs[g] = idx_dma(g, g % _NIBUF)

    def _zrow(r, carry):
        for b in range(_NBUF):
            for u in range(_CHUNK // _L):
                bufs[b][r, pl.ds(u * _L, _L)] = zeros16
        return carry

    lax.fori_loop(0, _CPW, _zrow, 0)

    def scan_scatter(b, ib, val16):
        def body(jj, carry):
            for u in range(4):
                col = jj * (4 * _L) + u * _L
                xv = ibufs[ib][pl.ds(col, _L)]
                cls = xv - lo
                msk = (cls >= 0) & (cls < _CPW)
                plsc.store_scatter(bufs[b], [cls, lane + col], val16,
                                   mask=msk)
            return carry
        lax.fori_loop(0, _CHUNK // (4 * _L), body, 0)

    def out_dma(b, g, nrows):
        src = bufs[b] if nrows == _CPW else bufs[b].at[pl.ds(0, nrows)]
        dst = out_hbm.at[pl.ds(lo, nrows), pl.ds(g * _CHUNK, _CHUNK)]
        return pltpu.async_copy(src, dst, sem_o[b])

    def pipeline(nrows):
        odescs = {}
        for g in range(_NCHUNKS):
            b = g % _NBUF
            ib = g % _NIBUF
            if g >= _NBUF:
                odescs[g - _NBUF].wait()
                pib = (g - _NBUF) % _NIBUF
                scan_scatter(b, pib, zeros16)
                if g - _NBUF + _NIBUF < _NCHUNKS:
                    idescs[g - _NBUF + _NIBUF] = idx_dma(
                        g - _NBUF + _NIBUF, pib)
            idescs[g].wait()
            scan_scatter(b, ib, ones16)
            odescs[g] = out_dma(b, g, nrows)
        for g in range(_NCHUNKS - _NBUF, _NCHUNKS):
            odescs[g].wait()

    @pl.when(wid < _NW - 1)
    def _():
        pipeline(_CPW)

    @pl.when(wid == _NW - 1)
    def _():
        pipeline(_C - _CPW * (_NW - 1))


_onehot_sc = functools.partial(
    pl.kernel,
    out_type=jax.ShapeDtypeStruct((_C, _N), jnp.int32),
    mesh=plsc.VectorSubcoreMesh(
        core_axis_name="c", subcore_axis_name="s",
        num_cores=_NC, num_subcores=_NS,
    ),
    scratch_types=(
        [pltpu.VMEM((_CPW, _CHUNK), jnp.int32) for _ in range(_NBUF)]
        + [pltpu.VMEM((_CHUNK,), jnp.int32) for _ in range(_NIBUF)]
        + [pltpu.SemaphoreType.DMA for _ in range(_NBUF + _NIBUF)]
    ),
    compiler_params=pltpu.CompilerParams(needs_layout_passes=False),
)(_onehot_body)


def kernel(x):
    return _onehot_sc(x).T


# speedup vs baseline: 3.6521x; 1.3198x over previous
import functools

import jax
import jax.numpy as jnp
from jax import lax
from jax.experimental import pallas as pl
from jax.experimental.pallas import tpu as pltpu
from jax.experimental.pallas import tpu_sc as plsc

_C = 1000
_N = 16384
_NC = 2
_NS = 16
_NW = _NC * _NS
_CPW = 32
_CHUNK = 1024
_NCHUNKS = _N // _CHUNK
_L = 16
_NBUF = 2


def _onehot_body(x_hbm, out_hbm, buf0, buf1, idx_v, sem_i, so0, so1):
    bufs = [buf0, buf1]
    sem_o = [so0, so1]

    cid = lax.axis_index("c")
    sid = lax.axis_index("s")
    wid = sid * _NC + cid
    lo = wid * _CPW

    zeros16 = jnp.zeros((_L,), jnp.int32)
    ones16 = jnp.ones((_L,), jnp.int32)
    lane = lax.iota(jnp.int32, _L)
    cpw_u = jnp.uint32(_CPW)

    idesc = pltpu.async_copy(x_hbm, idx_v, sem_i)

    def _zrow(r, carry):
        for b in range(_NBUF):
            for u in range(_CHUNK // _L):
                bufs[b][r, pl.ds(u * _L, _L)] = zeros16
        return carry

    lax.fori_loop(0, _CPW, _zrow, 0)
    idesc.wait()

    def scan_scatter(b, colbase, val16):
        def body(jj, carry):
            for u in range(4):
                rel = jj * (4 * _L) + u * _L
                xv = idx_v[pl.ds(colbase + rel, _L)]
                cls = xv - lo
                msk = plsc.bitcast(cls, jnp.uint32) < cpw_u
                plsc.store_scatter(bufs[b], [cls, lane + rel], val16,
                                   mask=msk)
            return carry
        lax.fori_loop(0, _CHUNK // (4 * _L), body, 0)

    def pipeline(nrows):
        def src(b):
            return bufs[b] if nrows == _CPW else bufs[b].at[pl.ds(0, nrows)]

        def dst(colbase):
            return out_hbm.at[pl.ds(lo, nrows),
                              pl.ds(pl.multiple_of(colbase, _CHUNK), _CHUNK)]

        def chunk_pair(p, carry):
            for b in range(_NBUF):
                g = p * _NBUF + b
                colbase = g * _CHUNK

                @pl.when(p > 0)
                def _():
                    prev = (g - _NBUF) * _CHUNK
                    pltpu.make_async_copy(src(b), dst(prev), sem_o[b]).wait()
                    scan_scatter(b, prev, zeros16)

                scan_scatter(b, colbase, ones16)
                pltpu.async_copy(src(b), dst(colbase), sem_o[b])
            return carry

        lax.fori_loop(0, _NCHUNKS // _NBUF, chunk_pair, 0)
        for b in range(_NBUF):
            g = _NCHUNKS - _NBUF + b
            pltpu.make_async_copy(src(b), dst(g * _CHUNK), sem_o[b]).wait()

    @pl.when(wid < _NW - 1)
    def _():
        pipeline(_CPW)

    @pl.when(wid == _NW - 1)
    def _():
        pipeline(_C - _CPW * (_NW - 1))


_onehot_sc = functools.partial(
    pl.kernel,
    out_type=jax.ShapeDtypeStruct((_C, _N), jnp.int32),
    mesh=plsc.VectorSubcoreMesh(
        core_axis_name="c", subcore_axis_name="s",
        num_cores=_NC, num_subcores=_NS,
    ),
    scratch_types=(
        [pltpu.VMEM((_CPW, _CHUNK), jnp.int32) for _ in range(_NBUF)]
        + [pltpu.VMEM((_N,), jnp.int32)]
        + [pltpu.SemaphoreType.DMA for _ in range(1 + _NBUF)]
    ),
    compiler_params=pltpu.CompilerParams(needs_layout_passes=False),
)(_onehot_body)


def kernel(x):
    return _onehot_sc(x).T
